# Initial kernel scaffold; baseline (speedup 1.0000x reference)
import jax
import jax.numpy as jnp
from jax.experimental import pallas as pl


def kernel(node_features, edges, W_hidden, b_hidden, W_att, b_att):
    raise NotImplementedError("write your pallas kernel here")


# speedup vs baseline: 6.0722x; 6.0722x over previous
import functools

import jax
import jax.numpy as jnp
from jax import lax
from jax.experimental import pallas as pl
from jax.experimental.pallas import tpu as pltpu
from jax.experimental.pallas import tpu_sc as plsc

NCORES = 2
NSUB = 16
NW = NCORES * NSUB
CH = 64
BLK = 64


def _tc_feat_body(x_ref, w_ref, wa_ref, bh_ref, ba_ref, feat_ref, ac_ref):
    f = jnp.dot(x_ref[...], w_ref[...], preferred_element_type=jnp.float32)
    f = f + bh_ref[...]
    feat_ref[...] = f
    ac_ref[...] = (
        jnp.dot(f, wa_ref[...], preferred_element_type=jnp.float32) + ba_ref[...]
    )


def _tc_combine_body(a0_ref, a1_ref, s0_ref, s1_ref, out_ref):
    s = s0_ref[...][:, 0:1] + s1_ref[...][:, 0:1]
    num = a0_ref[...] + a1_ref[...]
    out_ref[...] = jnp.where(s > 0.0, num / s, 0.0)


def _make_sc_kernel(npad, nsum, ntab, nch, hdim):
    mesh = plsc.VectorSubcoreMesh(
        core_axis_name="c", subcore_axis_name="s",
        num_cores=NCORES, num_subcores=NSUB,
    )
    rows_per_sub = npad // NSUB
    srows_per_sub = nsum // NSUB

    @functools.partial(
        pl.kernel,
        out_type=[
            jax.ShapeDtypeStruct(
                (NCORES, NSUB, rows_per_sub, hdim), jnp.float32),
            jax.ShapeDtypeStruct((NCORES * nsum,), jnp.float32),
        ],
        mesh=mesh,
        compiler_params=pltpu.CompilerParams(needs_layout_passes=False),
        scratch_types=[
            pltpu.VMEM((ntab,), jnp.float32),
            pltpu.VMEM((ntab,), jnp.float32),
            pltpu.VMEM((nch, CH), jnp.int32),
            pltpu.VMEM((nch, CH), jnp.int32),
            pltpu.VMEM((CH, hdim), jnp.float32),
            pltpu.VMEM((CH,), jnp.float32),
            pltpu.VMEM((640,), jnp.float32),
            pltpu.VMEM_SHARED((npad, hdim), jnp.float32),
            pltpu.VMEM_SHARED((nsum,), jnp.float32),
        ],
    )
    def sc_kernel(feat_hbm, a_hbm, c_hbm, src_hbm, dst_hbm,
                  acc_hbm, sums_hbm, a_v, c_v, src_v, dst_v, msg_v, p_v,
                  st_v, acc_s, sums_s):
        cid = lax.axis_index("c")
        sid = lax.axis_index("s")
        wid = cid * NSUB + sid

        if True:
            pltpu.sync_copy(a_hbm, a_v)
            pltpu.sync_copy(c_hbm, c_v)
            pltpu.sync_copy(src_hbm.at[wid], src_v)
            pltpu.sync_copy(dst_hbm.at[wid], dst_v)

            zero16 = jnp.zeros((16,), jnp.float32)
            for i in range(CH):
                for v in range(hdim // 16):
                    msg_v[i, pl.ds(v * 16, 16)] = zero16
            for i in range(640 // 16):
                st_v[pl.ds(i * 16, 16)] = zero16
            row0 = sid * rows_per_sub
            nfull, rem = divmod(rows_per_sub, CH)
            for k in range(nfull):
                pltpu.sync_copy(msg_v, acc_s.at[pl.ds(row0 + k * CH, CH)])
            if rem:
                pltpu.sync_copy(msg_v.at[pl.ds(0, rem)],
                                acc_s.at[pl.ds(row0 + nfull * CH, rem)])
            srow0 = sid * srows_per_sub
            pltpu.sync_copy(st_v.at[pl.ds(0, srows_per_sub)],
                            sums_s.at[pl.ds(srow0, srows_per_sub)])

            plsc.subcore_barrier()

            def chunk_body(ci, carry):
                pltpu.sync_copy(feat_hbm.at[dst_v.at[ci]], msg_v)

                for j in range(CH // 16):
                    s16 = src_v[ci, pl.ds(j * 16, 16)]
                    d16 = dst_v[ci, pl.ds(j * 16, 16)]
                    av = plsc.load_gather(a_v, [s16])
                    cv = plsc.load_gather(c_v, [d16])
                    x = av + cv
                    p16 = jnp.exp(jnp.maximum(x, 0.2 * x))
                    p_v[pl.ds(j * 16, 16)] = p16
                    for l in range(16):
                        i = j * 16 + l
                        ps = jnp.full((16,), p16[l], jnp.float32)
                        for v in range(hdim // 16):
                            sl = pl.ds(v * 16, 16)
                            msg_v[i, sl] = msg_v[i, sl] * ps

                pltpu.sync_copy(msg_v, acc_s.at[src_v.at[ci]], add=True)
                pltpu.sync_copy(p_v, sums_s.at[src_v.at[ci]], add=True)
                return carry

            lax.fori_loop(0, nch, chunk_body, 0)

            plsc.subcore_barrier()

            pltpu.sync_copy(acc_s.at[pl.ds(row0, rows_per_sub)],
                            acc_hbm.at[cid, sid])
            pltpu.sync_copy(sums_s.at[pl.ds(srow0, srows_per_sub)],
                            st_v.at[pl.ds(0, srows_per_sub)])
            pltpu.sync_copy(st_v.at[pl.ds(0, srows_per_sub)],
                            sums_hbm.at[pl.ds(cid * nsum + srow0, srows_per_sub)])

    return sc_kernel


def kernel(node_features, edges, W_hidden, b_hidden, W_att, b_att):
    n, d = node_features.shape
    h = W_hidden.shape[1]
    e = edges.shape[0]

    npad = ((n + 1 + BLK - 1) // BLK) * BLK
    epw = ((e + NW * CH - 1) // (NW * CH)) * (NW * CH) // NW
    nch = epw // CH
    epad = epw * NW

    xp = jnp.pad(node_features, ((0, npad - n), (0, 0)))
    wa = W_att.reshape(2, h).T
    bh2 = b_hidden.reshape(1, h)
    ba2 = jnp.concatenate([b_att, jnp.zeros((1,), jnp.float32)]).reshape(1, 2)
    pad_e = epad - e
    src_p = jnp.concatenate(
        [edges[:, 0], jnp.full((pad_e,), n, jnp.int32)]).reshape(NW, nch, CH)
    dst_p = jnp.concatenate(
        [edges[:, 1], jnp.full((pad_e,), n, jnp.int32)]).reshape(NW, nch, CH)

    feat, ac = pl.pallas_call(
        _tc_feat_body,
        grid=(npad // BLK,),
        in_specs=[
            pl.BlockSpec((BLK, d), lambda i: (i, 0)),
            pl.BlockSpec((d, h), lambda i: (0, 0)),
            pl.BlockSpec((h, 2), lambda i: (0, 0)),
            pl.BlockSpec((1, h), lambda i: (0, 0)),
            pl.BlockSpec((1, 2), lambda i: (0, 0)),
        ],
        out_specs=[
            pl.BlockSpec((BLK, h), lambda i: (i, 0)),
            pl.BlockSpec((BLK, 2), lambda i: (i, 0)),
        ],
        out_shape=[
            jax.ShapeDtypeStruct((npad, h), jnp.float32),
            jax.ShapeDtypeStruct((npad, 2), jnp.float32),
        ],
    )(xp, W_hidden, wa, bh2, ba2)

    ntab = ((n + 1 + 7) // 8) * 8
    nsum = ((n + 1 + 127) // 128) * 128
    acc, sums = _make_sc_kernel(npad, nsum, ntab, nch, h)(
        feat, ac[:ntab, 0], ac[:ntab, 1], src_p, dst_p)
    acc = acc.reshape(NCORES, npad, h)
    sums = sums.reshape(NCORES, nsum)[:, :npad, None]

    out = pl.pallas_call(
        _tc_combine_body,
        grid=(npad // BLK,),
        in_specs=[
            pl.BlockSpec((BLK, h), lambda i: (i, 0)),
            pl.BlockSpec((BLK, h), lambda i: (i, 0)),
            pl.BlockSpec((BLK, 1), lambda i: (i, 0)),
            pl.BlockSpec((BLK, 1), lambda i: (i, 0)),
        ],
        out_specs=pl.BlockSpec((BLK, h), lambda i: (i, 0)),
        out_shape=jax.ShapeDtypeStruct((npad, h), jnp.float32),
    )(acc[0], acc[1], sums[0], sums[1])

    return out[:n]
